# baseline (speedup 1.0000x reference)
import functools

import jax
import jax.numpy as jnp
from jax import lax
from jax.experimental import pallas as pl
from jax.experimental.pallas import tpu as pltpu
from jax.experimental.pallas import tpu_sc as plsc

VOCAB = 100000
HIDDEN = 1024
B, S = 4, 4096
N = B * S

_info = plsc.get_sparse_core_info()
_NC, _NS = _info.num_cores, _info.num_subcores
_NW = _NC * _NS
_BPW = N // _NW
_C = 8
_NBUF = 14
_NCHUNK = _BPW // _C
_AHEAD = 7

_mesh = plsc.VectorSubcoreMesh(core_axis_name="c", subcore_axis_name="s")


@functools.partial(
    pl.kernel,
    mesh=_mesh,
    out_type=jax.ShapeDtypeStruct((N, HIDDEN), jnp.float32),
    scratch_types=[
        pltpu.VMEM((_BPW,), jnp.int32),
    ]
    + [pltpu.VMEM((_C, HIDDEN), jnp.float32) for _ in range(_NBUF)]
    + [pltpu.SemaphoreType.DMA for _ in range(2 * _NBUF)],
)
def _emb_lookup(table_hbm, idx_hbm, out_hbm, idx_v, *bufs_and_sems):
    bufs = bufs_and_sems[:_NBUF]
    sems_g = bufs_and_sems[_NBUF:2 * _NBUF]
    sems_w = bufs_and_sems[2 * _NBUF:]

    wid = lax.axis_index("s") * _NC + lax.axis_index("c")
    base = wid * _BPW
    pltpu.sync_copy(idx_hbm.at[pl.ds(base, _BPW)], idx_v)

    def gather(g):
        b = g % _NBUF
        return pltpu.async_copy(
            table_hbm.at[idx_v.at[pl.ds(g * _C, _C)]], bufs[b], sems_g[b])

    def write(g):
        b = g % _NBUF
        return pltpu.async_copy(
            bufs[b], out_hbm.at[pl.ds(base + g * _C, _C)], sems_w[b])

    gh = [None] * _NBUF
    wh = [None] * _NBUF
    for g in range(_AHEAD):
        gh[g % _NBUF] = gather(g)
    for g in range(_NCHUNK):
        b = g % _NBUF
        if g + _AHEAD < _NCHUNK:
            nb = (g + _AHEAD) % _NBUF
            if wh[nb] is not None:
                wh[nb].wait()
            gh[nb] = gather(g + _AHEAD)
        gh[b].wait()
        wh[b] = write(g)
    for b in range(_NBUF):
        if wh[b] is not None:
            wh[b].wait()


def kernel(input_ids, table):
    flat_ids = input_ids.reshape(N).astype(jnp.int32)
    out = _emb_lookup(table, flat_ids)
    return out.reshape(B, S, HIDDEN)


# speedup vs baseline: 1.0256x; 1.0256x over previous
import functools

import jax
import jax.numpy as jnp
from jax import lax
from jax.experimental import pallas as pl
from jax.experimental.pallas import tpu as pltpu
from jax.experimental.pallas import tpu_sc as plsc

VOCAB = 100000
HIDDEN = 1024
B, S = 4, 4096
N = B * S

_info = plsc.get_sparse_core_info()
_NC, _NS = _info.num_cores, _info.num_subcores
_NW = _NC * _NS
_BPW = N // _NW
_WPR = S // _BPW
_C = 16
_NBUF = 7
_NCHUNK = _BPW // _C
_AHEAD = 4

_mesh = plsc.VectorSubcoreMesh(core_axis_name="c", subcore_axis_name="s")


@functools.partial(
    pl.kernel,
    mesh=_mesh,
    out_type=jax.ShapeDtypeStruct((B, S, HIDDEN), jnp.float32),
    scratch_types=[
        pltpu.VMEM((_BPW,), jnp.int32),
    ]
    + [pltpu.VMEM((_C, HIDDEN), jnp.float32) for _ in range(_NBUF)]
    + [pltpu.SemaphoreType.DMA for _ in range(2 * _NBUF)],
)
def _emb_lookup(table_hbm, idx_hbm, out_hbm, idx_v, *bufs_and_sems):
    bufs = bufs_and_sems[:_NBUF]
    sems_g = bufs_and_sems[_NBUF:2 * _NBUF]
    sems_w = bufs_and_sems[2 * _NBUF:]

    wid = lax.axis_index("s") * _NC + lax.axis_index("c")
    row = wid // _WPR
    col = (wid % _WPR) * _BPW
    pltpu.sync_copy(idx_hbm.at[row, pl.ds(col, _BPW)], idx_v)

    def gather(g):
        b = g % _NBUF
        return pltpu.async_copy(
            table_hbm.at[idx_v.at[pl.ds(g * _C, _C)]], bufs[b], sems_g[b])

    def write(g):
        b = g % _NBUF
        return pltpu.async_copy(
            bufs[b], out_hbm.at[row, pl.ds(col + g * _C, _C)], sems_w[b])

    gh = [None] * _NBUF
    wh = [None] * _NBUF
    for g in range(_AHEAD):
        gh[g % _NBUF] = gather(g)
    for g in range(_NCHUNK):
        b = g % _NBUF
        if g + _AHEAD < _NCHUNK:
            nb = (g + _AHEAD) % _NBUF
            if wh[nb] is not None:
                wh[nb].wait()
            gh[nb] = gather(g + _AHEAD)
        gh[b].wait()
        wh[b] = write(g)
    for b in range(_NBUF):
        if wh[b] is not None:
            wh[b].wait()


def kernel(input_ids, table):
    return _emb_lookup(table, input_ids)


# speedup vs baseline: 1.0292x; 1.0036x over previous
import functools

import jax
import jax.numpy as jnp
from jax import lax
from jax.experimental import pallas as pl
from jax.experimental.pallas import tpu as pltpu
from jax.experimental.pallas import tpu_sc as plsc

VOCAB = 100000
HIDDEN = 1024
B, S = 4, 4096
N = B * S

_info = plsc.get_sparse_core_info()
_NC, _NS = _info.num_cores, _info.num_subcores
_NW = _NC * _NS
_BPW = N // _NW
_WPR = S // _BPW
_C = 16
_NBUF = 7
_NCHUNK = _BPW // _C
_AHEAD = 4

_mesh = plsc.VectorSubcoreMesh(core_axis_name="c", subcore_axis_name="s")


@functools.partial(
    pl.kernel,
    mesh=_mesh,
    out_type=jax.ShapeDtypeStruct((B, S, HIDDEN), jnp.float32),
    scratch_types=[
        pltpu.VMEM((_BPW,), jnp.int32),
    ]
    + [pltpu.VMEM((_C, HIDDEN), jnp.float32) for _ in range(_NBUF)]
    + [pltpu.SemaphoreType.DMA for _ in range(2 * _NBUF)],
)
def _emb_lookup(table_hbm, idx_hbm, out_hbm, idx_v, *bufs_and_sems):
    bufs = bufs_and_sems[:_NBUF]
    sems_g = bufs_and_sems[_NBUF:2 * _NBUF]
    sems_w = bufs_and_sems[2 * _NBUF:]

    wid = lax.axis_index("c") * _NS + lax.axis_index("s")
    row = wid // _WPR
    col = (wid % _WPR) * _BPW
    pltpu.sync_copy(idx_hbm.at[row, pl.ds(col, _BPW)], idx_v)

    def gather(g):
        b = g % _NBUF
        return pltpu.async_copy(
            table_hbm.at[idx_v.at[pl.ds(g * _C, _C)]], bufs[b], sems_g[b])

    def write(g):
        b = g % _NBUF
        return pltpu.async_copy(
            bufs[b], out_hbm.at[row, pl.ds(col + g * _C, _C)], sems_w[b])

    gh = [None] * _NBUF
    wh = [None] * _NBUF
    for g in range(_AHEAD):
        gh[g % _NBUF] = gather(g)
    for g in range(_NCHUNK):
        b = g % _NBUF
        if g + _AHEAD < _NCHUNK:
            nb = (g + _AHEAD) % _NBUF
            if wh[nb] is not None:
                wh[nb].wait()
            gh[nb] = gather(g + _AHEAD)
        gh[b].wait()
        wh[b] = write(g)
    for b in range(_NBUF):
        if wh[b] is not None:
            wh[b].wait()


def kernel(input_ids, table):
    return _emb_lookup(table, input_ids)
